# baseline (speedup 1.0000x reference)
import functools

import jax
import jax.numpy as jnp
from jax import lax
from jax.experimental import pallas as pl
from jax.experimental.pallas import tpu as pltpu
from jax.experimental.pallas import tpu_sc as plsc

N_NODES = 10000
D = 128
N_EDGES = 320000

NC = 2
NS = 16
NW = NC * NS
CHUNK = 128
CPW = 80
NIB = 2
IBC = CPW // NIB
EPAD = NW * CPW * CHUNK
NPAD = 10112
RPT = NPAD // NS
BLK = 1264
NBUF = 2
KAHEAD = 1
_ZOFFS = sorted({*range(0, RPT - CHUNK + 1, CHUNK), RPT - CHUNK})


@functools.cache
def _build_deg_kernel():
    mesh = plsc.VectorSubcoreMesh(
        core_axis_name="c", subcore_axis_name="s", num_cores=NC, num_subcores=NS
    )
    return functools.partial(
        pl.kernel,
        out_type=jax.ShapeDtypeStruct((NC, NPAD, 16), jnp.float32),
        mesh=mesh,
        scratch_types=[
            pltpu.VMEM((CPW, CHUNK), jnp.int32),
            pltpu.VMEM((CHUNK, 16), jnp.float32),
            pltpu.VMEM((CHUNK, 16), jnp.float32),
            pltpu.VMEM_SHARED((NPAD, 16), jnp.float32),
        ],
    )(_deg_body)


def _deg_body(dst_hbm, out_hbm, dst_v, ones_v, zero_v, deg_sh):
    cid = lax.axis_index("c")
    sid = lax.axis_index("s")
    wid = cid * NS + sid
    one16 = jnp.ones((16,), jnp.float32)
    zero16 = jnp.zeros((16,), jnp.float32)

    def fill(i, c):
        ones_v[i, :] = one16
        zero_v[i, :] = zero16
        return c

    lax.fori_loop(0, CHUNK, fill, 0)
    for off in _ZOFFS:
        pltpu.sync_copy(zero_v, deg_sh.at[pl.ds(sid * RPT + off, CHUNK)])
    plsc.subcore_barrier()

    pltpu.sync_copy(dst_hbm.at[wid], dst_v)

    def body(j, c):
        pltpu.sync_copy(ones_v, deg_sh.at[dst_v.at[j]], add=True)
        return c

    lax.fori_loop(0, CPW, body, 0)
    plsc.subcore_barrier()
    pltpu.sync_copy(
        deg_sh.at[pl.ds(sid * RPT, RPT)], out_hbm.at[cid, pl.ds(sid * RPT, RPT)]
    )


@functools.cache
def _build_prop_kernel():
    mesh = plsc.VectorSubcoreMesh(
        core_axis_name="c", subcore_axis_name="s", num_cores=NC, num_subcores=NS
    )
    return functools.partial(
        pl.kernel,
        out_type=jax.ShapeDtypeStruct((NC, NPAD, D), jnp.float32),
        mesh=mesh,
        scratch_types=[
            pltpu.VMEM((IBC, CHUNK), jnp.int32),
            pltpu.VMEM((IBC, CHUNK), jnp.int32),
            pltpu.VMEM((NBUF, CHUNK, D), jnp.float32),
            pltpu.VMEM_SHARED((NPAD, D), jnp.float32),
            pltpu.SemaphoreType.DMA((NBUF,)),
        ],
    )(_prop_body)


def _prop_body(
    lin_hbm, src_hbm, dst_hbm, out_hbm, src_v, dst_v, rows_v, acc_sh, gsem
):
    cid = lax.axis_index("c")
    sid = lax.axis_index("s")
    wid = cid * NS + sid
    zero16 = jnp.zeros((16,), jnp.float32)

    def zfill(i, c):
        rows_v[0, i // 8, pl.ds((i % 8) * 16, 16)] = zero16
        return c

    lax.fori_loop(0, CHUNK * 8, zfill, 0)
    for off in _ZOFFS:
        pltpu.sync_copy(rows_v.at[0], acc_sh.at[pl.ds(sid * RPT + off, CHUNK)])
    plsc.subcore_barrier()

    for ib in range(NIB):
        pltpu.sync_copy(src_hbm.at[wid, pl.ds(ib * IBC, IBC)], src_v)
        pltpu.sync_copy(dst_hbm.at[wid, pl.ds(ib * IBC, IBC)], dst_v)

        for j in range(KAHEAD):
            pltpu.async_copy(
                lin_hbm.at[src_v.at[j]], rows_v.at[j % NBUF], gsem.at[j % NBUF]
            )

        def body(j, c):
            bs = lax.rem(j, NBUF)
            bg = lax.rem(j + KAHEAD, NBUF)

            @pl.when(j + KAHEAD < IBC)
            def _prefetch():
                pltpu.async_copy(
                    lin_hbm.at[src_v.at[j + KAHEAD]], rows_v.at[bg], gsem.at[bg]
                )

            pltpu.make_async_copy(
                lin_hbm.at[src_v.at[j]], rows_v.at[bs], gsem.at[bs]
            ).wait()
            pltpu.sync_copy(rows_v.at[bs], acc_sh.at[dst_v.at[j]], add=True)
            return c

        lax.fori_loop(0, IBC, body, 0)
    plsc.subcore_barrier()
    pltpu.sync_copy(
        acc_sh.at[pl.ds(sid * RPT, RPT)], out_hbm.at[cid, pl.ds(sid * RPT, RPT)]
    )


def _linear_scale(x, w, d0, d1):
    def body(x_ref, w_ref, d0_ref, d1_ref, dinv_ref, lins_ref):
        d = d0_ref[...] + d1_ref[...] + 1.0
        dinv = lax.rsqrt(d)
        dinv_ref[...] = dinv
        lin = jnp.dot(x_ref[...], w_ref[...], preferred_element_type=jnp.float32)
        lins_ref[...] = lin * dinv[:, 0:1]

    return pl.pallas_call(
        body,
        grid=(NPAD // BLK,),
        in_specs=[
            pl.BlockSpec((BLK, D), lambda i: (i, 0)),
            pl.BlockSpec((D, D), lambda i: (0, 0)),
            pl.BlockSpec((BLK, 16), lambda i: (i, 0)),
            pl.BlockSpec((BLK, 16), lambda i: (i, 0)),
        ],
        out_specs=[
            pl.BlockSpec((BLK, 16), lambda i: (i, 0)),
            pl.BlockSpec((BLK, D), lambda i: (i, 0)),
        ],
        out_shape=[
            jax.ShapeDtypeStruct((NPAD, 16), jnp.float32),
            jax.ShapeDtypeStruct((NPAD, D), jnp.float32),
        ],
    )(x, w, d0, d1)


def _mid_layer(p0, p1, lins, dinv, b, w):
    def body(p0_ref, p1_ref, l_ref, dv_ref, b_ref, w_ref, o_ref):
        dv = dv_ref[...][:, 0:1]
        h = (p0_ref[...] + p1_ref[...] + l_ref[...]) * dv + b_ref[...][None, :]
        h = jnp.maximum(h, 0.0)
        o_ref[...] = jnp.dot(h, w_ref[...], preferred_element_type=jnp.float32) * dv

    return pl.pallas_call(
        body,
        grid=(NPAD // BLK,),
        in_specs=[
            pl.BlockSpec((BLK, D), lambda i: (i, 0)),
            pl.BlockSpec((BLK, D), lambda i: (i, 0)),
            pl.BlockSpec((BLK, D), lambda i: (i, 0)),
            pl.BlockSpec((BLK, 16), lambda i: (i, 0)),
            pl.BlockSpec((D,), lambda i: (0,)),
            pl.BlockSpec((D, D), lambda i: (0, 0)),
        ],
        out_specs=pl.BlockSpec((BLK, D), lambda i: (i, 0)),
        out_shape=jax.ShapeDtypeStruct((NPAD, D), jnp.float32),
    )(p0, p1, lins, dinv, b, w)


def _final_layer(q0, q1, lins, dinv, b):
    def body(q0_ref, q1_ref, l_ref, dv_ref, b_ref, o_ref):
        dv = dv_ref[...][:, 0:1]
        o_ref[...] = (q0_ref[...] + q1_ref[...] + l_ref[...]) * dv + b_ref[...][None, :]

    return pl.pallas_call(
        body,
        grid=(NPAD // BLK,),
        in_specs=[
            pl.BlockSpec((BLK, D), lambda i: (i, 0)),
            pl.BlockSpec((BLK, D), lambda i: (i, 0)),
            pl.BlockSpec((BLK, D), lambda i: (i, 0)),
            pl.BlockSpec((BLK, 16), lambda i: (i, 0)),
            pl.BlockSpec((D,), lambda i: (0,)),
        ],
        out_specs=pl.BlockSpec((BLK, D), lambda i: (i, 0)),
        out_shape=jax.ShapeDtypeStruct((NPAD, D), jnp.float32),
    )(q0, q1, lins, dinv, b)


def kernel(x, edge_index, W1, b1, W2, b2):
    ei = edge_index.astype(jnp.int32)
    pad = EPAD - N_EDGES
    fill = jnp.full((pad,), N_NODES, jnp.int32)
    srcp = jnp.concatenate([ei[0], fill]).reshape(NW, CPW, CHUNK)
    dstp = jnp.concatenate([ei[1], fill]).reshape(NW, CPW, CHUNK)
    xp = jnp.pad(x, ((0, NPAD - N_NODES), (0, 0)))

    degp = _build_deg_kernel()(dstp)
    dinv, lins1 = _linear_scale(xp, W1, degp[0], degp[1])
    prop = _build_prop_kernel()
    p = prop(lins1, srcp, dstp)
    lins2 = _mid_layer(p[0], p[1], lins1, dinv, b1, W2)
    q = prop(lins2, srcp, dstp)
    outp = _final_layer(q[0], q[1], lins2, dinv, b2)
    return outp[:N_NODES]


# speedup vs baseline: 3.5654x; 3.5654x over previous
import functools

import jax
import jax.numpy as jnp
from jax import lax
from jax.experimental import pallas as pl
from jax.experimental.pallas import tpu as pltpu
from jax.experimental.pallas import tpu_sc as plsc

N_NODES = 10000
D = 128
N_EDGES = 320000

NC = 2
NS = 16
NW = NC * NS
CHUNK = 128
CPW = 80
NIB = 2
IBC = CPW // NIB
EPAD = NW * CPW * CHUNK
NPAD = 10112
RPT = NPAD // NS
BLK = 1264
NBUF = 2
KAHEAD = 1
_ZOFFS = sorted({*range(0, RPT - CHUNK + 1, CHUNK), RPT - CHUNK})


@functools.cache
def _build_deg_kernel():
    mesh = plsc.VectorSubcoreMesh(
        core_axis_name="c", subcore_axis_name="s", num_cores=NC, num_subcores=NS
    )
    return functools.partial(
        pl.kernel,
        out_type=jax.ShapeDtypeStruct((NC, NPAD, 16), jnp.float32),
        mesh=mesh,
        scratch_types=[
            pltpu.VMEM((CPW, CHUNK), jnp.int32),
            pltpu.VMEM((CHUNK, 16), jnp.float32),
            pltpu.VMEM((CHUNK, 16), jnp.float32),
            pltpu.VMEM_SHARED((NPAD, 16), jnp.float32),
        ],
    )(_deg_body)


def _deg_body(dst_hbm, out_hbm, dst_v, ones_v, zero_v, deg_sh):
    cid = lax.axis_index("c")
    sid = lax.axis_index("s")
    wid = cid * NS + sid
    one16 = jnp.ones((16,), jnp.float32)
    zero16 = jnp.zeros((16,), jnp.float32)

    def fill(i, c):
        ones_v[i, :] = one16
        zero_v[i, :] = zero16
        return c

    lax.fori_loop(0, CHUNK, fill, 0)
    for off in _ZOFFS:
        pltpu.sync_copy(zero_v, deg_sh.at[pl.ds(sid * RPT + off, CHUNK)])
    plsc.subcore_barrier()

    pltpu.sync_copy(dst_hbm.at[wid], dst_v)

    def body(j, c):
        pltpu.sync_copy(ones_v, deg_sh.at[dst_v.at[j]], add=True)
        return c

    lax.fori_loop(0, CPW, body, 0)
    plsc.subcore_barrier()
    pltpu.sync_copy(
        deg_sh.at[pl.ds(sid * RPT, RPT)], out_hbm.at[cid, pl.ds(sid * RPT, RPT)]
    )


@functools.cache
def _build_prop_kernel():
    mesh = plsc.VectorSubcoreMesh(
        core_axis_name="c", subcore_axis_name="s", num_cores=NC, num_subcores=NS
    )
    return functools.partial(
        pl.kernel,
        out_type=jax.ShapeDtypeStruct((NC, NPAD, D), jnp.float32),
        mesh=mesh,
        scratch_types=[
            pltpu.VMEM((IBC, CHUNK), jnp.int32),
            pltpu.VMEM((IBC, CHUNK), jnp.int32),
            pltpu.VMEM((NBUF, CHUNK, D), jnp.float32),
            pltpu.VMEM_SHARED((NPAD, D), jnp.float32),
            pltpu.SemaphoreType.DMA((NBUF,)),
        ],
    )(_prop_body)


def _prop_body(
    lin_hbm, src_hbm, dst_hbm, out_hbm, src_v, dst_v, rows_v, acc_sh, gsem
):
    cid = lax.axis_index("c")
    sid = lax.axis_index("s")
    wid = cid * NS + sid
    zero16 = jnp.zeros((16,), jnp.float32)

    def zfill(i, c):
        rows_v[0, i // 8, pl.ds((i % 8) * 16, 16)] = zero16
        return c

    lax.fori_loop(0, CHUNK * 8, zfill, 0)
    for off in _ZOFFS:
        pltpu.sync_copy(rows_v.at[0], acc_sh.at[pl.ds(sid * RPT + off, CHUNK)])
    plsc.subcore_barrier()

    for ib in range(NIB):
        pltpu.sync_copy(src_hbm.at[wid, pl.ds(ib * IBC, IBC)], src_v)
        pltpu.sync_copy(dst_hbm.at[wid, pl.ds(ib * IBC, IBC)], dst_v)

        for j in range(KAHEAD):
            pltpu.async_copy(
                lin_hbm.at[src_v.at[j]], rows_v.at[j % NBUF], gsem.at[j % NBUF]
            )

        def body(j, c):
            bs = lax.rem(j, NBUF)
            bg = lax.rem(j + KAHEAD, NBUF)

            @pl.when(j + KAHEAD < IBC)
            def _prefetch():
                pltpu.async_copy(
                    lin_hbm.at[src_v.at[j + KAHEAD]], rows_v.at[bg], gsem.at[bg]
                )

            pltpu.make_async_copy(
                lin_hbm.at[src_v.at[j]], rows_v.at[bs], gsem.at[bs]
            ).wait()
            pltpu.sync_copy(rows_v.at[bs], acc_sh.at[dst_v.at[j]], add=True)
            return c

        lax.fori_loop(0, IBC, body, 0)
    plsc.subcore_barrier()
    pltpu.sync_copy(
        acc_sh.at[pl.ds(sid * RPT, RPT)], out_hbm.at[cid, pl.ds(sid * RPT, RPT)]
    )


def _linear_scale(x, w, d0, d1):
    def body(x_ref, w_ref, d0_ref, d1_ref, dinv_ref, lins_ref):
        d = d0_ref[...] + d1_ref[...] + 1.0
        dinv = lax.rsqrt(d)
        dinv_ref[...] = dinv
        lin = jnp.dot(x_ref[...], w_ref[...], preferred_element_type=jnp.float32)
        lins_ref[...] = lin * dinv[:, 0:1]

    return pl.pallas_call(
        body,
        grid=(NPAD // BLK,),
        in_specs=[
            pl.BlockSpec((BLK, D), lambda i: (i, 0)),
            pl.BlockSpec((D, D), lambda i: (0, 0)),
            pl.BlockSpec((BLK, 16), lambda i: (i, 0)),
            pl.BlockSpec((BLK, 16), lambda i: (i, 0)),
        ],
        out_specs=[
            pl.BlockSpec((BLK, 16), lambda i: (i, 0)),
            pl.BlockSpec((BLK, D), lambda i: (i, 0)),
        ],
        out_shape=[
            jax.ShapeDtypeStruct((NPAD, 16), jnp.float32),
            jax.ShapeDtypeStruct((NPAD, D), jnp.float32),
        ],
    )(x, w, d0, d1)


def _mid_layer(p0, p1, lins, dinv, b, w):
    def body(p0_ref, p1_ref, l_ref, dv_ref, b_ref, w_ref, o_ref):
        dv = dv_ref[...][:, 0:1]
        h = (p0_ref[...] + p1_ref[...] + l_ref[...]) * dv + b_ref[...][None, :]
        h = jnp.maximum(h, 0.0)
        o_ref[...] = jnp.dot(h, w_ref[...], preferred_element_type=jnp.float32) * dv

    return pl.pallas_call(
        body,
        grid=(NPAD // BLK,),
        in_specs=[
            pl.BlockSpec((BLK, D), lambda i: (i, 0)),
            pl.BlockSpec((BLK, D), lambda i: (i, 0)),
            pl.BlockSpec((BLK, D), lambda i: (i, 0)),
            pl.BlockSpec((BLK, 16), lambda i: (i, 0)),
            pl.BlockSpec((D,), lambda i: (0,)),
            pl.BlockSpec((D, D), lambda i: (0, 0)),
        ],
        out_specs=pl.BlockSpec((BLK, D), lambda i: (i, 0)),
        out_shape=jax.ShapeDtypeStruct((NPAD, D), jnp.float32),
    )(p0, p1, lins, dinv, b, w)


def _final_layer(q0, q1, lins, dinv, b):
    def body(q0_ref, q1_ref, l_ref, dv_ref, b_ref, o_ref):
        dv = dv_ref[...][:, 0:1]
        o_ref[...] = (q0_ref[...] + q1_ref[...] + l_ref[...]) * dv + b_ref[...][None, :]

    return pl.pallas_call(
        body,
        grid=(NPAD // BLK,),
        in_specs=[
            pl.BlockSpec((BLK, D), lambda i: (i, 0)),
            pl.BlockSpec((BLK, D), lambda i: (i, 0)),
            pl.BlockSpec((BLK, D), lambda i: (i, 0)),
            pl.BlockSpec((BLK, 16), lambda i: (i, 0)),
            pl.BlockSpec((D,), lambda i: (0,)),
        ],
        out_specs=pl.BlockSpec((BLK, D), lambda i: (i, 0)),
        out_shape=jax.ShapeDtypeStruct((NPAD, D), jnp.float32),
    )(q0, q1, lins, dinv, b)


def kernel(x, edge_index, W1, b1, W2, b2):
    ei = edge_index.astype(jnp.int32)
    pad = EPAD - N_EDGES
    fill = N_NODES + jnp.arange(pad, dtype=jnp.int32) % (NPAD - N_NODES)
    srcp = jnp.concatenate([ei[0], fill]).reshape(NW, CPW, CHUNK)
    dstp = jnp.concatenate([ei[1], fill]).reshape(NW, CPW, CHUNK)
    xp = jnp.pad(x, ((0, NPAD - N_NODES), (0, 0)))

    degp = _build_deg_kernel()(dstp)
    dinv, lins1 = _linear_scale(xp, W1, degp[0], degp[1])
    prop = _build_prop_kernel()
    p = prop(lins1, srcp, dstp)
    lins2 = _mid_layer(p[0], p[1], lins1, dinv, b1, W2)
    q = prop(lins2, srcp, dstp)
    outp = _final_layer(q[0], q[1], lins2, dinv, b2)
    return outp[:N_NODES]


# speedup vs baseline: 3.9068x; 1.0957x over previous
import functools

import jax
import jax.numpy as jnp
from jax import lax
from jax.experimental import pallas as pl
from jax.experimental.pallas import tpu as pltpu
from jax.experimental.pallas import tpu_sc as plsc

N_NODES = 10000
D = 128
N_EDGES = 320000

NC = 2
NS = 16
NW = NC * NS
CHUNK = 128
CPW = 80
NIB = 2
IBC = CPW // NIB
EPAD = NW * CPW * CHUNK
NPAD = 10112
RPT = NPAD // NS
BLK = 1264
NBUF = 2
KAHEAD = 1
_ZOFFS = sorted({*range(0, RPT - CHUNK + 1, CHUNK), RPT - CHUNK})


@functools.cache
def _build_deg_kernel():
    mesh = plsc.VectorSubcoreMesh(
        core_axis_name="c", subcore_axis_name="s", num_cores=NC, num_subcores=NS
    )
    return functools.partial(
        pl.kernel,
        out_type=jax.ShapeDtypeStruct((NC, NPAD, 16), jnp.float32),
        mesh=mesh,
        scratch_types=[
            pltpu.VMEM((CPW, CHUNK), jnp.int32),
            pltpu.VMEM((CHUNK, 16), jnp.float32),
            pltpu.VMEM((CHUNK, 16), jnp.float32),
            pltpu.VMEM_SHARED((NPAD, 16), jnp.float32),
        ],
    )(_deg_body)


def _deg_body(dst_hbm, out_hbm, dst_v, ones_v, zero_v, deg_sh):
    cid = lax.axis_index("c")
    sid = lax.axis_index("s")
    wid = cid * NS + sid
    one16 = jnp.ones((16,), jnp.float32)
    zero16 = jnp.zeros((16,), jnp.float32)

    def fill(i, c):
        ones_v[i, :] = one16
        zero_v[i, :] = zero16
        return c

    lax.fori_loop(0, CHUNK, fill, 0)
    for off in _ZOFFS:
        pltpu.sync_copy(zero_v, deg_sh.at[pl.ds(sid * RPT + off, CHUNK)])
    plsc.subcore_barrier()

    pltpu.sync_copy(dst_hbm.at[wid], dst_v)

    def body(j, c):
        pltpu.sync_copy(ones_v, deg_sh.at[dst_v.at[j]], add=True)
        return c

    lax.fori_loop(0, CPW, body, 0)
    plsc.subcore_barrier()
    pltpu.sync_copy(
        deg_sh.at[pl.ds(sid * RPT, RPT)], out_hbm.at[cid, pl.ds(sid * RPT, RPT)]
    )


@functools.cache
def _build_prop_kernel():
    mesh = plsc.VectorSubcoreMesh(
        core_axis_name="c", subcore_axis_name="s", num_cores=NC, num_subcores=NS
    )
    return functools.partial(
        pl.kernel,
        out_type=jax.ShapeDtypeStruct((NC, NPAD, D), jnp.float32),
        mesh=mesh,
        scratch_types=[
            pltpu.VMEM((IBC, CHUNK), jnp.int32),
            pltpu.VMEM((IBC, CHUNK), jnp.int32),
            pltpu.VMEM((NBUF, CHUNK, D), jnp.float32),
            pltpu.VMEM_SHARED((NPAD, D), jnp.float32),
            pltpu.SemaphoreType.DMA((NBUF,)),
        ],
    )(_prop_body)


def _prop_body(
    lin_hbm, src_hbm, dst_hbm, out_hbm, src_v, dst_v, rows_v, acc_sh, gsem
):
    cid = lax.axis_index("c")
    sid = lax.axis_index("s")
    wid = cid * NS + sid
    zero16 = jnp.zeros((16,), jnp.float32)

    def zfill(i, c):
        rows_v[0, i // 8, pl.ds((i % 8) * 16, 16)] = zero16
        return c

    lax.fori_loop(0, CHUNK * 8, zfill, 0)
    for off in _ZOFFS:
        pltpu.sync_copy(rows_v.at[0], acc_sh.at[pl.ds(sid * RPT + off, CHUNK)])
    plsc.subcore_barrier()

    for ib in range(NIB):
        pltpu.sync_copy(src_hbm.at[wid, pl.ds(ib * IBC, IBC)], src_v)
        pltpu.sync_copy(dst_hbm.at[wid, pl.ds(ib * IBC, IBC)], dst_v)

        for j in range(KAHEAD):
            pltpu.async_copy(
                lin_hbm.at[src_v.at[j]], rows_v.at[j % NBUF], gsem.at[j % NBUF]
            )

        def body(j, c):
            bs = lax.rem(j, NBUF)
            bg = lax.rem(j + KAHEAD, NBUF)

            @pl.when(j + KAHEAD < IBC)
            def _prefetch():
                pltpu.async_copy(
                    lin_hbm.at[src_v.at[j + KAHEAD]], rows_v.at[bg], gsem.at[bg]
                )

            pltpu.make_async_copy(
                lin_hbm.at[src_v.at[j]], rows_v.at[bs], gsem.at[bs]
            ).wait()
            pltpu.sync_copy(rows_v.at[bs], acc_sh.at[dst_v.at[j]], add=True)
            return c

        lax.fori_loop(0, IBC, body, 0)
    plsc.subcore_barrier()
    pltpu.sync_copy(
        acc_sh.at[pl.ds(sid * RPT, RPT)], out_hbm.at[cid, pl.ds(sid * RPT, RPT)]
    )


TBLK = 2000
NTB = N_NODES // TBLK


def _matmul(x, w):
    def body(x_ref, w_ref, o_ref):
        o_ref[...] = jnp.dot(x_ref[...], w_ref[...], preferred_element_type=jnp.float32)

    return pl.pallas_call(
        body,
        grid=(NTB,),
        in_specs=[
            pl.BlockSpec((TBLK, D), lambda i: (i, 0)),
            pl.BlockSpec((D, D), lambda i: (0, 0)),
        ],
        out_specs=pl.BlockSpec((TBLK, D), lambda i: (i, 0)),
        out_shape=jax.ShapeDtypeStruct((N_NODES, D), jnp.float32),
    )(x, w)


def _scale(lin, degp):
    def body(l_ref, dg_ref, dinv_ref, lins_ref):
        d = dg_ref[0] + dg_ref[1] + 1.0
        dinv = lax.rsqrt(d)
        dinv_ref[...] = dinv
        lins_ref[...] = l_ref[...] * dinv[:, 0:1]

    return pl.pallas_call(
        body,
        grid=(NTB,),
        in_specs=[
            pl.BlockSpec((TBLK, D), lambda i: (i, 0)),
            pl.BlockSpec((NC, TBLK, 16), lambda i: (0, i, 0)),
        ],
        out_specs=[
            pl.BlockSpec((TBLK, 16), lambda i: (i, 0)),
            pl.BlockSpec((TBLK, D), lambda i: (i, 0)),
        ],
        out_shape=[
            jax.ShapeDtypeStruct((N_NODES, 16), jnp.float32),
            jax.ShapeDtypeStruct((N_NODES, D), jnp.float32),
        ],
    )(lin, degp)


def _mid_layer(q, lins, dinv, b, w):
    def body(q_ref, l_ref, dv_ref, b_ref, w_ref, o_ref):
        dv = dv_ref[...][:, 0:1]
        h = (q_ref[0] + q_ref[1] + l_ref[...]) * dv + b_ref[...][None, :]
        h = jnp.maximum(h, 0.0)
        o_ref[...] = jnp.dot(h, w_ref[...], preferred_element_type=jnp.float32) * dv

    return pl.pallas_call(
        body,
        grid=(NTB,),
        in_specs=[
            pl.BlockSpec((NC, TBLK, D), lambda i: (0, i, 0)),
            pl.BlockSpec((TBLK, D), lambda i: (i, 0)),
            pl.BlockSpec((TBLK, 16), lambda i: (i, 0)),
            pl.BlockSpec((D,), lambda i: (0,)),
            pl.BlockSpec((D, D), lambda i: (0, 0)),
        ],
        out_specs=pl.BlockSpec((TBLK, D), lambda i: (i, 0)),
        out_shape=jax.ShapeDtypeStruct((N_NODES, D), jnp.float32),
    )(q, lins, dinv, b, w)


def _final_layer(q, lins, dinv, b):
    def body(q_ref, l_ref, dv_ref, b_ref, o_ref):
        dv = dv_ref[...][:, 0:1]
        o_ref[...] = (q_ref[0] + q_ref[1] + l_ref[...]) * dv + b_ref[...][None, :]

    return pl.pallas_call(
        body,
        grid=(NTB,),
        in_specs=[
            pl.BlockSpec((NC, TBLK, D), lambda i: (0, i, 0)),
            pl.BlockSpec((TBLK, D), lambda i: (i, 0)),
            pl.BlockSpec((TBLK, 16), lambda i: (i, 0)),
            pl.BlockSpec((D,), lambda i: (0,)),
        ],
        out_specs=pl.BlockSpec((TBLK, D), lambda i: (i, 0)),
        out_shape=jax.ShapeDtypeStruct((N_NODES, D), jnp.float32),
    )(q, lins, dinv, b)


def kernel(x, edge_index, W1, b1, W2, b2):
    ei = edge_index.astype(jnp.int32)
    pad = EPAD - N_EDGES
    fill_dst = N_NODES + jnp.arange(pad, dtype=jnp.int32) % (NPAD - N_NODES)
    fill_src = jnp.arange(pad, dtype=jnp.int32) % N_NODES
    srcp = jnp.concatenate([ei[0], fill_src]).reshape(NW, CPW, CHUNK)
    dstp = jnp.concatenate([ei[1], fill_dst]).reshape(NW, CPW, CHUNK)

    lin1 = _matmul(x, W1)
    degp = _build_deg_kernel()(dstp)
    dinv, lins1 = _scale(lin1, degp)
    prop = _build_prop_kernel()
    q1 = prop(lins1, srcp, dstp)
    lins2 = _mid_layer(q1, lins1, dinv, b1, W2)
    q2 = prop(lins2, srcp, dstp)
    return _final_layer(q2, lins2, dinv, b2)
